# Initial kernel scaffold; baseline (speedup 1.0000x reference)
import jax
import jax.numpy as jnp
from jax.experimental import pallas as pl


def kernel(nodes, children, w_t, w_l, w_r, b):
    raise NotImplementedError("write your pallas kernel here")


# speedup vs baseline: 24.8092x; 24.8092x over previous
import jax
import jax.numpy as jnp
from jax import lax
from jax.experimental import pallas as pl
from jax.experimental.pallas import tpu as pltpu
from jax.experimental.pallas import tpu_sc as plsc

_B, _N, _C, _F, _O = 16, 2048, 8, 128, 128
_NC, _NS, _L = 2, 16, 16
_NW = _NC * _NS
_BN = _B * _N
_CHUNK = _BN // _NW
_G = 8
_K = _C * _G
_NG = _CHUNK // _G
_FV = _F // _L


def _sc_body(nodes_hbm, ch_hbm, ur_hbm, ul_hbm,
             ch_v, coef_r, coef_l, idx_v,
             rows0, rows1, outr0, outr1, outl0, outl1,
             gsem0, gsem1, orsem0, orsem1, olsem0, olsem1):
  cid = lax.axis_index("c")
  sid = lax.axis_index("s")
  base = sid * _N + cid * _CHUNK
  rowbase = sid * _N

  for c in range(_C):
    pltpu.sync_copy(ch_hbm.at[c, pl.ds(base, _CHUNK)], ch_v.at[c])

  def _coef_body(q, carry):
    off = q * _L
    chs, mfs = [], []
    ns = jnp.zeros((_L,), jnp.float32)
    for c in range(_C):
      ch = ch_v[c, pl.ds(off, _L)]
      mf = jnp.where(ch > 0, 1.0, 0.0).astype(jnp.float32)
      chs.append(ch)
      mfs.append(mf)
      ns = ns + mf
    is1 = ns == 1.0
    inv = 1.0 / jnp.where(is1, 1.0, ns - 1.0)
    for c in range(_C):
      sel = jnp.where(is1, 0.5 if c == 0 else 0.0, float(c) * mfs[c] * inv)
      cr = sel * mfs[c]
      cl = (1.0 - sel) * mfs[c]
      gi = chs[c] + rowbase
      coef_r[2 * q, pl.ds(c * _G, _G)] = cr[0:_G]
      coef_r[2 * q + 1, pl.ds(c * _G, _G)] = cr[_G:_L]
      coef_l[2 * q, pl.ds(c * _G, _G)] = cl[0:_G]
      coef_l[2 * q + 1, pl.ds(c * _G, _G)] = cl[_G:_L]
      idx_v[2 * q, pl.ds(c * _G, _G)] = gi[0:_G]
      idx_v[2 * q + 1, pl.ds(c * _G, _G)] = gi[_G:_L]
    return carry

  lax.fori_loop(0, _CHUNK // _L, _coef_body, 0)

  rows = (rows0, rows1)
  outr = (outr0, outr1)
  outl = (outl0, outl1)
  gsem = (gsem0, gsem1)
  orsem = (orsem0, orsem1)
  olsem = (olsem0, olsem1)

  def _start_gather(g, j):
    pltpu.async_copy(nodes_hbm.at[idx_v.at[g]], rows[j], gsem[j])

  def _wait_gather(g, j):
    pltpu.make_async_copy(nodes_hbm.at[idx_v.at[g]], rows[j], gsem[j]).wait()

  def _start_out(g, j):
    dst = pl.ds(base + g * _G, _G)
    pltpu.async_copy(outr[j], ur_hbm.at[dst], orsem[j])
    pltpu.async_copy(outl[j], ul_hbm.at[dst], olsem[j])

  def _wait_out(g, j):
    dst = pl.ds(base + g * _G, _G)
    pltpu.make_async_copy(outr[j], ur_hbm.at[dst], orsem[j]).wait()
    pltpu.make_async_copy(outl[j], ul_hbm.at[dst], olsem[j]).wait()

  _start_gather(0, 0)

  def _outer(g2, carry):
    for j in range(2):
      g = g2 + j

      @pl.when(g + 1 < _NG)
      def _():
        _start_gather(g + 1, 1 - j)

      _wait_gather(g, j)

      @pl.when(g2 > 0)
      def _():
        _wait_out(g - 2, j)

      crv = [coef_r[g, pl.ds(k * _L, _L)] for k in range(_K // _L)]
      clv = [coef_l[g, pl.ds(k * _L, _L)] for k in range(_K // _L)]
      for n in range(_G):
        accr = [None] * _FV
        accl = [None] * _FV
        for c in range(_C):
          p = c * _G + n
          wr = crv[p // _L][p % _L]
          wl = clv[p // _L][p % _L]
          for f in range(_FV):
            v = rows[j][p, pl.ds(f * _L, _L)]
            if c == 0:
              accr[f] = wr * v
              accl[f] = wl * v
            else:
              accr[f] = accr[f] + wr * v
              accl[f] = accl[f] + wl * v
        for f in range(_FV):
          outr[j][n, pl.ds(f * _L, _L)] = accr[f]
          outl[j][n, pl.ds(f * _L, _L)] = accl[f]
      _start_out(g, j)
    return carry

  lax.fori_loop(0, _NG // 2, lambda i, c: _outer(i * 2, c), 0)

  for j in range(2):
    _wait_out(_NG - 2 + j, j)


def _sc_gather(nodes_flat, ch_cm):
  mesh = plsc.VectorSubcoreMesh(core_axis_name="c", subcore_axis_name="s",
                                num_cores=_NC, num_subcores=_NS)
  f32 = jnp.float32
  return pl.kernel(
      _sc_body,
      out_type=[jax.ShapeDtypeStruct((_BN, _F), f32),
                jax.ShapeDtypeStruct((_BN, _F), f32)],
      mesh=mesh,
      scratch_types=[
          pltpu.VMEM((_C, _CHUNK), jnp.int32),
          pltpu.VMEM((_NG, _K), f32),
          pltpu.VMEM((_NG, _K), f32),
          pltpu.VMEM((_NG, _K), jnp.int32),
          pltpu.VMEM((_K, _F), f32),
          pltpu.VMEM((_K, _F), f32),
          pltpu.VMEM((_G, _F), f32),
          pltpu.VMEM((_G, _F), f32),
          pltpu.VMEM((_G, _F), f32),
          pltpu.VMEM((_G, _F), f32),
          pltpu.SemaphoreType.DMA,
          pltpu.SemaphoreType.DMA,
          pltpu.SemaphoreType.DMA,
          pltpu.SemaphoreType.DMA,
          pltpu.SemaphoreType.DMA,
          pltpu.SemaphoreType.DMA,
      ],
  )(nodes_flat, ch_cm)


def _tc_body(x_ref, ur_ref, ul_ref, m0_ref, m1_ref, m2_ref, b_ref, o_ref):
  acc = jnp.dot(x_ref[...], m0_ref[...], preferred_element_type=jnp.float32)
  acc = acc + jnp.dot(ur_ref[...], m1_ref[...], preferred_element_type=jnp.float32)
  acc = acc + jnp.dot(ul_ref[...], m2_ref[...], preferred_element_type=jnp.float32)
  o_ref[...] = jnp.maximum(acc + b_ref[...], 0.0)


def _tc_matmul(nodes_flat, u_r, u_l, m0, m1, m2, b2):
  blk = 2048
  grid = (_BN // blk,)
  row_spec = pl.BlockSpec((blk, _F), lambda i: (i, 0))
  w_spec = pl.BlockSpec((_F, _O), lambda i: (0, 0))
  return pl.pallas_call(
      _tc_body,
      grid=grid,
      in_specs=[row_spec, row_spec, row_spec, w_spec, w_spec, w_spec,
                pl.BlockSpec((1, _O), lambda i: (0, 0))],
      out_specs=pl.BlockSpec((blk, _O), lambda i: (i, 0)),
      out_shape=jax.ShapeDtypeStruct((_BN, _O), jnp.float32),
  )(nodes_flat, u_r, u_l, m0, m1, m2, b2)


def kernel(nodes, children, w_t, w_l, w_r, b):
  nodes_flat = nodes.reshape(_BN, _F)
  ch_cm = children.transpose(2, 0, 1).reshape(_C, _BN)
  wflat = jnp.stack([w_t, w_r, w_l]).reshape(3 * _F, _O).reshape(_F, 3, _O)
  m0 = wflat[:, 0, :]
  m1 = wflat[:, 1, :]
  m2 = wflat[:, 2, :]
  u_r, u_l = _sc_gather(nodes_flat, ch_cm)
  out = _tc_matmul(nodes_flat, u_r, u_l, m0, m1, m2, b.reshape(1, _O))
  return out.reshape(_B, _N, _O)
